# Initial kernel scaffold; baseline (speedup 1.0000x reference)
import jax
import jax.numpy as jnp
from jax.experimental import pallas as pl


def kernel(events, W, b):
    raise NotImplementedError("write your pallas kernel here")


# speedup vs baseline: 15.9054x; 15.9054x over previous
import functools

import jax
import jax.numpy as jnp
from jax import lax
from jax.experimental import pallas as pl
from jax.experimental.pallas import tpu as pltpu
from jax.experimental.pallas import tpu_sc as plsc

E_EVENTS = 4
N_PTS = 4096
D_FEAT = 16
K_NBR = 16
D_OUT = 64

R_BLK = 256

SC_NC = 2
SC_NS = 16
SC_NW = SC_NC * SC_NS
SC_LANES = 16

NP_TOT = E_EVENTS * N_PTS
PW = NP_TOT // SC_NW
P_CHUNK = 8
N_CHUNK = PW // P_CHUNK


def _tc_body(ev_ref, ct_ref, wt_ref, bf_ref, am_ref, idx_ref):
    e = pl.program_id(0)
    rb = pl.program_id(1)
    start = rb * R_BLK
    eb = ev_ref[0, pl.ds(start, R_BLK), :]
    xq = ev_ref[0, pl.ds(start, R_BLK), 0:1]
    yq = ev_ref[0, pl.ds(start, R_BLK), 1:2]
    xk = ct_ref[0, 0:1, :]
    yk = ct_ref[0, 1:2, :]
    dx = xq - xk
    dy = yq - yk
    d = jnp.sqrt(dx * dx + dy * dy)
    col = lax.broadcasted_iota(jnp.int32, (R_BLK, N_PTS), 1)
    row = lax.broadcasted_iota(jnp.int32, (R_BLK, N_PTS), 0) + start
    d = jnp.where(col == row, jnp.inf, d)
    idxs = []
    for _ in range(K_NBR):
        m = jnp.min(d, axis=1, keepdims=True)
        cand = jnp.where(d == m, col, N_PTS)
        amin = jnp.min(cand, axis=1, keepdims=True)
        idxs.append(amin)
        d = jnp.where(cand == amin, jnp.inf, d)
    gidx = jnp.concatenate(idxs, axis=1) + e * N_PTS
    idx_ref[0] = gidx
    am_ref[0] = (
        jnp.dot(eb, wt_ref[...], preferred_element_type=jnp.float32) + bf_ref[...]
    )


def _tc_call_kwargs():
    return dict(
        grid=(E_EVENTS, N_PTS // R_BLK),
        in_specs=[
            pl.BlockSpec((1, N_PTS, D_FEAT), lambda e, rb: (e, 0, 0)),
            pl.BlockSpec((1, 2, N_PTS), lambda e, rb: (e, 0, 0)),
            pl.BlockSpec((D_FEAT, 2 * D_OUT), lambda e, rb: (0, 0)),
            pl.BlockSpec((1, 2 * D_OUT), lambda e, rb: (0, 0)),
        ],
        out_specs=[
            pl.BlockSpec((1, R_BLK, 2 * D_OUT), lambda e, rb: (e, rb, 0)),
            pl.BlockSpec((1, R_BLK, K_NBR), lambda e, rb: (e, rb, 0)),
        ],
        out_shape=[
            jax.ShapeDtypeStruct((E_EVENTS, N_PTS, 2 * D_OUT), jnp.float32),
            jax.ShapeDtypeStruct((E_EVENTS, N_PTS, K_NBR), jnp.int32),
        ],
    )


def _sc_body(idx_hbm, am_hbm, out_hbm, idx_v, rows_v, a_v, o_v, sem):
    wid = lax.axis_index("s") * SC_NC + lax.axis_index("c")

    def chunk(c, carry):
        base = wid * PW + c * P_CHUNK
        pltpu.sync_copy(idx_hbm.at[pl.ds(base * K_NBR, P_CHUNK * K_NBR)], idx_v)
        pltpu.sync_copy(am_hbm.at[pl.ds(base, P_CHUNK)], a_v)
        pltpu.async_copy(am_hbm.at[idx_v], rows_v, sem).wait()

        def particle(p, carry2):
            accs = [jnp.zeros((SC_LANES,), jnp.float32)
                    for _ in range(D_OUT // SC_LANES)]
            for j in range(K_NBR):
                r = p * K_NBR + j
                for q in range(D_OUT // SC_LANES):
                    aq = a_v[p, pl.ds(q * SC_LANES, SC_LANES)]
                    accs[q] = accs[q] + jnp.maximum(
                        aq + rows_v[r, pl.ds(D_OUT + q * SC_LANES, SC_LANES)],
                        0.0,
                    )
            for q in range(D_OUT // SC_LANES):
                o_v[p, pl.ds(q * SC_LANES, SC_LANES)] = accs[q] * (1.0 / K_NBR)
            return carry2

        lax.fori_loop(0, P_CHUNK, particle, 0)
        pltpu.sync_copy(o_v, out_hbm.at[pl.ds(base, P_CHUNK)])
        return carry

    lax.fori_loop(0, N_CHUNK, chunk, 0)


def _sc_call_kwargs():
    return dict(
        out_type=jax.ShapeDtypeStruct((NP_TOT, D_OUT), jnp.float32),
        mesh=plsc.VectorSubcoreMesh(
            core_axis_name="c", subcore_axis_name="s",
            num_cores=SC_NC, num_subcores=SC_NS,
        ),
        scratch_types=[
            pltpu.VMEM((P_CHUNK * K_NBR,), jnp.int32),
            pltpu.VMEM((P_CHUNK * K_NBR, 2 * D_OUT), jnp.float32),
            pltpu.VMEM((P_CHUNK, 2 * D_OUT), jnp.float32),
            pltpu.VMEM((P_CHUNK, D_OUT), jnp.float32),
            pltpu.SemaphoreType.DMA,
        ],
    )


def kernel(events, W, b):
    W1 = W[:, :D_FEAT]
    W2 = W[:, D_FEAT:]
    wt = jnp.concatenate([(W1 - W2).T, W2.T], axis=1)
    bf = jnp.concatenate([b, jnp.zeros_like(b)]).reshape(1, 2 * D_OUT)
    coords_t = jnp.transpose(events[:, :, 0:2], (0, 2, 1))

    am, idx = pl.pallas_call(_tc_body, **_tc_call_kwargs())(
        events, coords_t, wt, bf
    )
    am_flat = am.reshape(NP_TOT, 2 * D_OUT)
    idx_flat = idx.reshape(NP_TOT * K_NBR)

    sc = functools.partial(pl.kernel, **_sc_call_kwargs())(_sc_body)
    out = sc(idx_flat, am_flat)
    return out.reshape(E_EVENTS, N_PTS, D_OUT)


# speedup vs baseline: 20.0740x; 1.2621x over previous
import functools

import jax
import jax.numpy as jnp
from jax import lax
from jax.experimental import pallas as pl
from jax.experimental.pallas import tpu as pltpu
from jax.experimental.pallas import tpu_sc as plsc

E_EVENTS = 4
N_PTS = 4096
D_FEAT = 16
K_NBR = 16
D_OUT = 64

R_BLK = 256

SC_NC = 2
SC_NS = 16
SC_NW = SC_NC * SC_NS
SC_LANES = 16

NP_TOT = E_EVENTS * N_PTS
PW = NP_TOT // SC_NW
P_CHUNK = 8
N_CHUNK = PW // P_CHUNK


def _tc_body(ev_ref, ct_ref, wt_ref, bf_ref, am_ref, idx_ref):
    e = pl.program_id(0)
    rb = pl.program_id(1)
    start = rb * R_BLK
    eb = ev_ref[0, pl.ds(start, R_BLK), :]
    xq = ev_ref[0, pl.ds(start, R_BLK), 0:1]
    yq = ev_ref[0, pl.ds(start, R_BLK), 1:2]
    xk = ct_ref[0, 0:1, :]
    yk = ct_ref[0, 1:2, :]
    dx = xq - xk
    dy = yq - yk
    d = dx * dx + dy * dy
    colf = lax.broadcasted_iota(jnp.int32, (R_BLK, N_PTS), 1).astype(jnp.float32)
    rowf = lax.broadcasted_iota(jnp.int32, (R_BLK, N_PTS), 0).astype(
        jnp.float32
    ) + start.astype(jnp.float32)
    d = jnp.where(colf == rowf, jnp.inf, d)
    idxs = []
    for _ in range(K_NBR):
        m = jnp.min(d, axis=1, keepdims=True)
        cand = jnp.where(d == m, colf, float(N_PTS))
        amin = jnp.min(cand, axis=1, keepdims=True)
        idxs.append(amin)
        d = jnp.where(cand == amin, jnp.inf, d)
    gidx = (
        jnp.concatenate(idxs, axis=1).astype(jnp.int32) + e * N_PTS
    )
    idx_ref[0] = gidx
    am_ref[0] = (
        jnp.dot(eb, wt_ref[...], preferred_element_type=jnp.float32) + bf_ref[...]
    )


def _tc_call_kwargs():
    return dict(
        grid=(E_EVENTS, N_PTS // R_BLK),
        in_specs=[
            pl.BlockSpec((1, N_PTS, D_FEAT), lambda e, rb: (e, 0, 0)),
            pl.BlockSpec((1, 2, N_PTS), lambda e, rb: (e, 0, 0)),
            pl.BlockSpec((D_FEAT, 2 * D_OUT), lambda e, rb: (0, 0)),
            pl.BlockSpec((1, 2 * D_OUT), lambda e, rb: (0, 0)),
        ],
        out_specs=[
            pl.BlockSpec((1, R_BLK, 2 * D_OUT), lambda e, rb: (e, rb, 0)),
            pl.BlockSpec((1, R_BLK, K_NBR), lambda e, rb: (e, rb, 0)),
        ],
        out_shape=[
            jax.ShapeDtypeStruct((E_EVENTS, N_PTS, 2 * D_OUT), jnp.float32),
            jax.ShapeDtypeStruct((E_EVENTS, N_PTS, K_NBR), jnp.int32),
        ],
    )


def _sc_body(idx_hbm, am_hbm, out_hbm, idx_v, rows_v, a_v, o_v, sem):
    wid = lax.axis_index("s") * SC_NC + lax.axis_index("c")

    def chunk(c, carry):
        base = wid * PW + c * P_CHUNK
        pltpu.sync_copy(idx_hbm.at[pl.ds(base * K_NBR, P_CHUNK * K_NBR)], idx_v)
        pltpu.sync_copy(am_hbm.at[pl.ds(base, P_CHUNK)], a_v)
        pltpu.async_copy(am_hbm.at[idx_v], rows_v, sem).wait()

        def particle(p, carry2):
            accs = [jnp.zeros((SC_LANES,), jnp.float32)
                    for _ in range(D_OUT // SC_LANES)]
            for j in range(K_NBR):
                r = p * K_NBR + j
                for q in range(D_OUT // SC_LANES):
                    aq = a_v[p, pl.ds(q * SC_LANES, SC_LANES)]
                    accs[q] = accs[q] + jnp.maximum(
                        aq + rows_v[r, pl.ds(D_OUT + q * SC_LANES, SC_LANES)],
                        0.0,
                    )
            for q in range(D_OUT // SC_LANES):
                o_v[p, pl.ds(q * SC_LANES, SC_LANES)] = accs[q] * (1.0 / K_NBR)
            return carry2

        lax.fori_loop(0, P_CHUNK, particle, 0)
        pltpu.sync_copy(o_v, out_hbm.at[pl.ds(base, P_CHUNK)])
        return carry

    lax.fori_loop(0, N_CHUNK, chunk, 0)


def _sc_call_kwargs():
    return dict(
        out_type=jax.ShapeDtypeStruct((NP_TOT, D_OUT), jnp.float32),
        mesh=plsc.VectorSubcoreMesh(
            core_axis_name="c", subcore_axis_name="s",
            num_cores=SC_NC, num_subcores=SC_NS,
        ),
        scratch_types=[
            pltpu.VMEM((P_CHUNK * K_NBR,), jnp.int32),
            pltpu.VMEM((P_CHUNK * K_NBR, 2 * D_OUT), jnp.float32),
            pltpu.VMEM((P_CHUNK, 2 * D_OUT), jnp.float32),
            pltpu.VMEM((P_CHUNK, D_OUT), jnp.float32),
            pltpu.SemaphoreType.DMA,
        ],
    )


def kernel(events, W, b):
    W1 = W[:, :D_FEAT]
    W2 = W[:, D_FEAT:]
    wt = jnp.concatenate([(W1 - W2).T, W2.T], axis=1)
    bf = jnp.concatenate([b, jnp.zeros_like(b)]).reshape(1, 2 * D_OUT)
    coords_t = jnp.transpose(events[:, :, 0:2], (0, 2, 1))

    am, idx = pl.pallas_call(_tc_body, **_tc_call_kwargs())(
        events, coords_t, wt, bf
    )
    am_flat = am.reshape(NP_TOT, 2 * D_OUT)
    idx_flat = idx.reshape(NP_TOT * K_NBR)

    sc = functools.partial(pl.kernel, **_sc_call_kwargs())(_sc_body)
    out = sc(idx_flat, am_flat)
    return out.reshape(E_EVENTS, N_PTS, D_OUT)


# speedup vs baseline: 22.3810x; 1.1149x over previous
import functools

import jax
import jax.numpy as jnp
from jax import lax
from jax.experimental import pallas as pl
from jax.experimental.pallas import tpu as pltpu
from jax.experimental.pallas import tpu_sc as plsc

E_EVENTS = 4
N_PTS = 4096
D_FEAT = 16
K_NBR = 16
D_OUT = 64

R_BLK = 256

SC_NC = 2
SC_NS = 16
SC_NW = SC_NC * SC_NS
SC_LANES = 16

PW = N_PTS // SC_NW
P_CHUNK = 8
N_CHUNK = PW // P_CHUNK


def _tc_body(ev_ref, ct_ref, wt_ref, bf_ref, am_ref, idx_ref):
    rb = pl.program_id(0)
    start = rb * R_BLK
    eb = ev_ref[pl.ds(start, R_BLK), :]
    xq = ev_ref[pl.ds(start, R_BLK), 0:1]
    yq = ev_ref[pl.ds(start, R_BLK), 1:2]
    xk = ct_ref[0:1, :]
    yk = ct_ref[1:2, :]
    dx = xq - xk
    dy = yq - yk
    d = dx * dx + dy * dy
    colf = lax.broadcasted_iota(jnp.int32, (R_BLK, N_PTS), 1).astype(jnp.float32)
    rowf = lax.broadcasted_iota(jnp.int32, (R_BLK, N_PTS), 0).astype(
        jnp.float32
    ) + start.astype(jnp.float32)
    d = jnp.where(colf == rowf, jnp.inf, d)
    idxs = []
    for _ in range(K_NBR):
        m = jnp.min(d, axis=1, keepdims=True)
        cand = jnp.where(d == m, colf, float(N_PTS))
        amin = jnp.min(cand, axis=1, keepdims=True)
        idxs.append(amin)
        d = jnp.where(cand == amin, jnp.inf, d)
    gidx = jnp.concatenate(idxs, axis=1).astype(jnp.int32)
    idx_ref[...] = gidx
    am_ref[...] = (
        jnp.dot(eb, wt_ref[...], preferred_element_type=jnp.float32) + bf_ref[...]
    )


def _tc_call_kwargs():
    return dict(
        grid=(N_PTS // R_BLK,),
        in_specs=[
            pl.BlockSpec((N_PTS, D_FEAT), lambda rb: (0, 0)),
            pl.BlockSpec((2, N_PTS), lambda rb: (0, 0)),
            pl.BlockSpec((D_FEAT, 2 * D_OUT), lambda rb: (0, 0)),
            pl.BlockSpec((1, 2 * D_OUT), lambda rb: (0, 0)),
        ],
        out_specs=[
            pl.BlockSpec((R_BLK, 2 * D_OUT), lambda rb: (rb, 0)),
            pl.BlockSpec((R_BLK, K_NBR), lambda rb: (rb, 0)),
        ],
        out_shape=[
            jax.ShapeDtypeStruct((N_PTS, 2 * D_OUT), jnp.float32),
            jax.ShapeDtypeStruct((N_PTS, K_NBR), jnp.int32),
        ],
    )


def _sc_body(idx_hbm, am_hbm, out_hbm, idx_v, rows_v, a_v, o_v, sem):
    wid = lax.axis_index("s") * SC_NC + lax.axis_index("c")

    def chunk(c, carry):
        base = wid * PW + c * P_CHUNK
        pltpu.sync_copy(idx_hbm.at[pl.ds(base * K_NBR, P_CHUNK * K_NBR)], idx_v)
        pltpu.sync_copy(am_hbm.at[pl.ds(base, P_CHUNK)], a_v)
        pltpu.async_copy(am_hbm.at[idx_v], rows_v, sem).wait()

        def particle(p, carry2):
            accs = [jnp.zeros((SC_LANES,), jnp.float32)
                    for _ in range(D_OUT // SC_LANES)]
            for j in range(K_NBR):
                r = p * K_NBR + j
                for q in range(D_OUT // SC_LANES):
                    aq = a_v[p, pl.ds(q * SC_LANES, SC_LANES)]
                    accs[q] = accs[q] + jnp.maximum(
                        aq + rows_v[r, pl.ds(D_OUT + q * SC_LANES, SC_LANES)],
                        0.0,
                    )
            for q in range(D_OUT // SC_LANES):
                o_v[p, pl.ds(q * SC_LANES, SC_LANES)] = accs[q] * (1.0 / K_NBR)
            return carry2

        lax.fori_loop(0, P_CHUNK, particle, 0)
        pltpu.sync_copy(o_v, out_hbm.at[pl.ds(base, P_CHUNK)])
        return carry

    lax.fori_loop(0, N_CHUNK, chunk, 0)


def _sc_call_kwargs():
    return dict(
        out_type=jax.ShapeDtypeStruct((N_PTS, D_OUT), jnp.float32),
        mesh=plsc.VectorSubcoreMesh(
            core_axis_name="c", subcore_axis_name="s",
            num_cores=SC_NC, num_subcores=SC_NS,
        ),
        scratch_types=[
            pltpu.VMEM((P_CHUNK * K_NBR,), jnp.int32),
            pltpu.VMEM((P_CHUNK * K_NBR, 2 * D_OUT), jnp.float32),
            pltpu.VMEM((P_CHUNK, 2 * D_OUT), jnp.float32),
            pltpu.VMEM((P_CHUNK, D_OUT), jnp.float32),
            pltpu.SemaphoreType.DMA,
        ],
    )


def kernel(events, W, b):
    W1 = W[:, :D_FEAT]
    W2 = W[:, D_FEAT:]
    wt = jnp.concatenate([(W1 - W2).T, W2.T], axis=1)
    bf = jnp.concatenate([b, jnp.zeros_like(b)]).reshape(1, 2 * D_OUT)
    coords_t = jnp.transpose(events[:, :, 0:2], (0, 2, 1))

    tc = pl.pallas_call(_tc_body, **_tc_call_kwargs())
    sc = functools.partial(pl.kernel, **_sc_call_kwargs())(_sc_body)

    outs = []
    for e in range(E_EVENTS):
        am, idx = tc(events[e], coords_t[e], wt, bf)
        outs.append(sc(idx.reshape(N_PTS * K_NBR), am))
    return jnp.stack(outs)


# speedup vs baseline: 37.2664x; 1.6651x over previous
import functools

import jax
import jax.numpy as jnp
from jax import lax
from jax.experimental import pallas as pl
from jax.experimental.pallas import tpu as pltpu
from jax.experimental.pallas import tpu_sc as plsc

E_EVENTS = 4
N_PTS = 4096
D_FEAT = 16
K_NBR = 16
D_OUT = 64

R_BLK = 256

SC_NC = 2
SC_NS = 16
SC_NW = SC_NC * SC_NS
SC_LANES = 16

PW = N_PTS // SC_NW
P_CHUNK = 8
N_CHUNK = PW // P_CHUNK


def _tc_body(ev_ref, ct_ref, wt_ref, bf_ref, am_ref, idx_ref):
    rb = pl.program_id(0)
    start = rb * R_BLK
    eb = ev_ref[pl.ds(start, R_BLK), :]
    xq = ev_ref[pl.ds(start, R_BLK), 0:1]
    yq = ev_ref[pl.ds(start, R_BLK), 1:2]
    xk = ct_ref[0:1, :]
    yk = ct_ref[1:2, :]
    dx = xq - xk
    dy = yq - yk
    d = dx * dx + dy * dy
    col = lax.broadcasted_iota(jnp.int32, (R_BLK, N_PTS), 1)
    row = lax.broadcasted_iota(jnp.int32, (R_BLK, N_PTS), 0) + start
    d = jnp.where(col == row, 3.0e38, d)
    keyi = (lax.bitcast_convert_type(d, jnp.int32) & ~0xFFF) | col
    key = lax.bitcast_convert_type(keyi, jnp.float32)
    f = jnp.minimum(key[:, : N_PTS // 2], key[:, N_PTS // 2 :])
    s = jnp.maximum(key[:, : N_PTS // 2], key[:, N_PTS // 2 :])
    ms = []
    for _ in range(K_NBR):
        m = jnp.min(f, axis=1, keepdims=True)
        hit = f == m
        f = jnp.where(hit, s, f)
        s = jnp.where(hit, 3.3e38, s)
        ms.append(m)
    gidx = (
        lax.bitcast_convert_type(jnp.concatenate(ms, axis=1), jnp.int32) & 0xFFF
    )
    idx_ref[...] = gidx
    am_ref[...] = (
        jnp.dot(eb, wt_ref[...], preferred_element_type=jnp.float32) + bf_ref[...]
    )


def _tc_call_kwargs():
    return dict(
        grid=(N_PTS // R_BLK,),
        in_specs=[
            pl.BlockSpec((N_PTS, D_FEAT), lambda rb: (0, 0)),
            pl.BlockSpec((2, N_PTS), lambda rb: (0, 0)),
            pl.BlockSpec((D_FEAT, 2 * D_OUT), lambda rb: (0, 0)),
            pl.BlockSpec((1, 2 * D_OUT), lambda rb: (0, 0)),
        ],
        out_specs=[
            pl.BlockSpec((R_BLK, 2 * D_OUT), lambda rb: (rb, 0)),
            pl.BlockSpec((R_BLK, K_NBR), lambda rb: (rb, 0)),
        ],
        out_shape=[
            jax.ShapeDtypeStruct((N_PTS, 2 * D_OUT), jnp.float32),
            jax.ShapeDtypeStruct((N_PTS, K_NBR), jnp.int32),
        ],
    )


def _sc_body(idx_hbm, am_hbm, out_hbm, idx_v, rows_v, a_v, o_v, sem):
    wid = lax.axis_index("s") * SC_NC + lax.axis_index("c")

    def chunk(c, carry):
        base = wid * PW + c * P_CHUNK
        pltpu.sync_copy(idx_hbm.at[pl.ds(base * K_NBR, P_CHUNK * K_NBR)], idx_v)
        pltpu.sync_copy(am_hbm.at[pl.ds(base, P_CHUNK)], a_v)
        pltpu.async_copy(am_hbm.at[idx_v], rows_v, sem).wait()

        def particle(p, carry2):
            accs = [jnp.zeros((SC_LANES,), jnp.float32)
                    for _ in range(D_OUT // SC_LANES)]
            for j in range(K_NBR):
                r = p * K_NBR + j
                for q in range(D_OUT // SC_LANES):
                    aq = a_v[p, pl.ds(q * SC_LANES, SC_LANES)]
                    accs[q] = accs[q] + jnp.maximum(
                        aq + rows_v[r, pl.ds(D_OUT + q * SC_LANES, SC_LANES)],
                        0.0,
                    )
            for q in range(D_OUT // SC_LANES):
                o_v[p, pl.ds(q * SC_LANES, SC_LANES)] = accs[q] * (1.0 / K_NBR)
            return carry2

        lax.fori_loop(0, P_CHUNK, particle, 0)
        pltpu.sync_copy(o_v, out_hbm.at[pl.ds(base, P_CHUNK)])
        return carry

    lax.fori_loop(0, N_CHUNK, chunk, 0)


def _sc_call_kwargs():
    return dict(
        out_type=jax.ShapeDtypeStruct((N_PTS, D_OUT), jnp.float32),
        mesh=plsc.VectorSubcoreMesh(
            core_axis_name="c", subcore_axis_name="s",
            num_cores=SC_NC, num_subcores=SC_NS,
        ),
        scratch_types=[
            pltpu.VMEM((P_CHUNK * K_NBR,), jnp.int32),
            pltpu.VMEM((P_CHUNK * K_NBR, 2 * D_OUT), jnp.float32),
            pltpu.VMEM((P_CHUNK, 2 * D_OUT), jnp.float32),
            pltpu.VMEM((P_CHUNK, D_OUT), jnp.float32),
            pltpu.SemaphoreType.DMA,
        ],
    )


def kernel(events, W, b):
    W1 = W[:, :D_FEAT]
    W2 = W[:, D_FEAT:]
    wt = jnp.concatenate([(W1 - W2).T, W2.T], axis=1)
    bf = jnp.concatenate([b, jnp.zeros_like(b)]).reshape(1, 2 * D_OUT)
    coords_t = jnp.transpose(events[:, :, 0:2], (0, 2, 1))

    tc = pl.pallas_call(_tc_body, **_tc_call_kwargs())
    sc = functools.partial(pl.kernel, **_sc_call_kwargs())(_sc_body)

    outs = []
    for e in range(E_EVENTS):
        am, idx = tc(events[e], coords_t[e], wt, bf)
        outs.append(sc(idx.reshape(N_PTS * K_NBR), am))
    return jnp.stack(outs)


# speedup vs baseline: 37.2767x; 1.0003x over previous
import functools

import jax
import jax.numpy as jnp
from jax import lax
from jax.experimental import pallas as pl
from jax.experimental.pallas import tpu as pltpu
from jax.experimental.pallas import tpu_sc as plsc

E_EVENTS = 4
N_PTS = 4096
D_FEAT = 16
K_NBR = 16
D_OUT = 64

R_BLK = 256

SC_NC = 2
SC_NS = 16
SC_NW = SC_NC * SC_NS
SC_LANES = 16

PW = N_PTS // SC_NW
P_CHUNK = 8
N_CHUNK = PW // P_CHUNK


def _tc_body(ev_ref, ct_ref, wt_ref, bf_ref, am_ref, idx_ref):
    rb = pl.program_id(0)
    start = rb * R_BLK
    eb = ev_ref[pl.ds(start, R_BLK), :]
    xq = ev_ref[pl.ds(start, R_BLK), 0:1]
    yq = ev_ref[pl.ds(start, R_BLK), 1:2]
    xk = ct_ref[0:1, :]
    yk = ct_ref[1:2, :]
    dx = xq - xk
    dy = yq - yk
    d = dx * dx + dy * dy
    col = lax.broadcasted_iota(jnp.int32, (R_BLK, N_PTS), 1)
    row = lax.broadcasted_iota(jnp.int32, (R_BLK, N_PTS), 0) + start
    d = jnp.where(col == row, 3.0e38, d)
    keyi = (lax.bitcast_convert_type(d, jnp.int32) & ~0xFFF) | col
    key = lax.bitcast_convert_type(keyi, jnp.float32)
    f = jnp.minimum(key[:, : N_PTS // 2], key[:, N_PTS // 2 :])
    s = jnp.maximum(key[:, : N_PTS // 2], key[:, N_PTS // 2 :])
    ms = []
    for _ in range(K_NBR):
        m = jnp.min(f, axis=1, keepdims=True)
        hit = f == m
        f = jnp.where(hit, s, f)
        s = jnp.where(hit, 3.3e38, s)
        ms.append(m)
    gidx = (
        lax.bitcast_convert_type(jnp.concatenate(ms, axis=1), jnp.int32) & 0xFFF
    )
    idx_ref[...] = gidx
    am_ref[...] = (
        jnp.dot(eb, wt_ref[...], preferred_element_type=jnp.float32) + bf_ref[...]
    )


def _tc_call_kwargs():
    return dict(
        grid=(N_PTS // R_BLK,),
        in_specs=[
            pl.BlockSpec((N_PTS, D_FEAT), lambda rb: (0, 0)),
            pl.BlockSpec((2, N_PTS), lambda rb: (0, 0)),
            pl.BlockSpec((D_FEAT, 2 * D_OUT), lambda rb: (0, 0)),
            pl.BlockSpec((1, 2 * D_OUT), lambda rb: (0, 0)),
        ],
        out_specs=[
            pl.BlockSpec((R_BLK, 2 * D_OUT), lambda rb: (rb, 0)),
            pl.BlockSpec((R_BLK, K_NBR), lambda rb: (rb, 0)),
        ],
        out_shape=[
            jax.ShapeDtypeStruct((N_PTS, 2 * D_OUT), jnp.float32),
            jax.ShapeDtypeStruct((N_PTS, K_NBR), jnp.int32),
        ],
    )


def _sc_body(idx_hbm, am_hbm, out_hbm, idx_v, rows_v, a_v, o_v, sem):
    wid = lax.axis_index("s") * SC_NC + lax.axis_index("c")

    def chunk(c, carry):
        base = wid * PW + c * P_CHUNK
        pltpu.sync_copy(idx_hbm.at[pl.ds(base * K_NBR, P_CHUNK * K_NBR)], idx_v)
        pltpu.sync_copy(am_hbm.at[pl.ds(base, P_CHUNK)], a_v)
        pltpu.async_copy(am_hbm.at[idx_v], rows_v, sem).wait()

        def particle(p, carry2):
            accs = [jnp.zeros((SC_LANES,), jnp.float32)
                    for _ in range(D_OUT // SC_LANES)]
            for j in range(K_NBR):
                r = p * K_NBR + j
                for q in range(D_OUT // SC_LANES):
                    aq = a_v[p, pl.ds(q * SC_LANES, SC_LANES)]
                    accs[q] = accs[q] + jnp.maximum(
                        aq + rows_v[r, pl.ds(D_OUT + q * SC_LANES, SC_LANES)],
                        0.0,
                    )
            for q in range(D_OUT // SC_LANES):
                o_v[p, pl.ds(q * SC_LANES, SC_LANES)] = accs[q] * (1.0 / K_NBR)
            return carry2

        lax.fori_loop(0, P_CHUNK, particle, 0)
        pltpu.sync_copy(o_v, out_hbm.at[pl.ds(base, P_CHUNK)])
        return carry

    lax.fori_loop(0, N_CHUNK, chunk, 0)


def _sc_call_kwargs():
    return dict(
        out_type=jax.ShapeDtypeStruct((N_PTS, D_OUT), jnp.float32),
        mesh=plsc.VectorSubcoreMesh(
            core_axis_name="c", subcore_axis_name="s",
            num_cores=SC_NC, num_subcores=SC_NS,
        ),
        scratch_types=[
            pltpu.VMEM((P_CHUNK * K_NBR,), jnp.int32),
            pltpu.VMEM((P_CHUNK * K_NBR, 2 * D_OUT), jnp.float32),
            pltpu.VMEM((P_CHUNK, 2 * D_OUT), jnp.float32),
            pltpu.VMEM((P_CHUNK, D_OUT), jnp.float32),
            pltpu.SemaphoreType.DMA,
        ],
    )


def kernel(events, W, b):
    W1 = W[:, :D_FEAT]
    W2 = W[:, D_FEAT:]
    wt = jnp.concatenate([(W1 - W2).T, W2.T], axis=1)
    bf = jnp.concatenate([b, jnp.zeros_like(b)]).reshape(1, 2 * D_OUT)
    coords_t = jnp.transpose(events[:, :, 0:2], (0, 2, 1))

    tc = pl.pallas_call(_tc_body, **_tc_call_kwargs())
    sc = functools.partial(pl.kernel, **_sc_call_kwargs())(_sc_body)

    tc_outs = [tc(events[e], coords_t[e], wt, bf) for e in range(E_EVENTS)]
    outs = [sc(idx.reshape(N_PTS * K_NBR), am) for am, idx in tc_outs]
    return jnp.stack(outs)


# speedup vs baseline: 38.8370x; 1.0419x over previous
import functools

import jax
import jax.numpy as jnp
from jax import lax
from jax.experimental import pallas as pl
from jax.experimental.pallas import tpu as pltpu
from jax.experimental.pallas import tpu_sc as plsc

E_EVENTS = 4
N_PTS = 4096
D_FEAT = 16
K_NBR = 16
D_OUT = 64

R_BLK = 256

SC_NC = 2
SC_NS = 16
SC_NW = SC_NC * SC_NS
SC_LANES = 16

PW = N_PTS // SC_NW
P_CHUNK = 8
N_CHUNK = PW // P_CHUNK


def _tc_body(ev_ref, ct_ref, wt_ref, bf_ref, am_ref, idx_ref):
    rb = pl.program_id(0)
    start = rb * R_BLK
    eb = ev_ref[pl.ds(start, R_BLK), :]
    xq = ev_ref[pl.ds(start, R_BLK), 0:1]
    yq = ev_ref[pl.ds(start, R_BLK), 1:2]
    xk = ct_ref[0:1, :]
    yk = ct_ref[1:2, :]
    dx = xq - xk
    dy = yq - yk
    d = dx * dx + dy * dy
    col = lax.broadcasted_iota(jnp.int32, (R_BLK, N_PTS), 1)
    row = lax.broadcasted_iota(jnp.int32, (R_BLK, N_PTS), 0) + start
    d = jnp.where(col == row, 3.0e38, d)
    keyi = (lax.bitcast_convert_type(d, jnp.int32) & ~0xFFF) | col
    key = lax.bitcast_convert_type(keyi, jnp.float32)
    f = jnp.minimum(key[:, : N_PTS // 2], key[:, N_PTS // 2 :])
    s = jnp.maximum(key[:, : N_PTS // 2], key[:, N_PTS // 2 :])
    ms = []
    for _ in range(K_NBR):
        m = jnp.min(f, axis=1, keepdims=True)
        hit = f == m
        f = jnp.where(hit, s, f)
        s = jnp.where(hit, 3.3e38, s)
        ms.append(m)
    gidx = (
        lax.bitcast_convert_type(jnp.concatenate(ms, axis=1), jnp.int32) & 0xFFF
    )
    idx_ref[...] = gidx
    am_ref[...] = (
        jnp.dot(eb, wt_ref[...], preferred_element_type=jnp.float32) + bf_ref[...]
    )


def _tc_call_kwargs():
    return dict(
        grid=(N_PTS // R_BLK,),
        in_specs=[
            pl.BlockSpec((N_PTS, D_FEAT), lambda rb: (0, 0)),
            pl.BlockSpec((2, N_PTS), lambda rb: (0, 0)),
            pl.BlockSpec((D_FEAT, 2 * D_OUT), lambda rb: (0, 0)),
            pl.BlockSpec((1, 2 * D_OUT), lambda rb: (0, 0)),
        ],
        out_specs=[
            pl.BlockSpec((R_BLK, 2 * D_OUT), lambda rb: (rb, 0)),
            pl.BlockSpec((R_BLK, K_NBR), lambda rb: (rb, 0)),
        ],
        out_shape=[
            jax.ShapeDtypeStruct((N_PTS, 2 * D_OUT), jnp.float32),
            jax.ShapeDtypeStruct((N_PTS, K_NBR), jnp.int32),
        ],
    )


G_IDX = P_CHUNK * K_NBR


def _sc_body(idx_hbm, am_hbm, out_hbm, idx_v, rows_v, a_v, o_v, sem0, sem1):
    wid = lax.axis_index("s") * SC_NC + lax.axis_index("c")
    pbase = wid * PW
    pltpu.sync_copy(idx_hbm.at[pl.ds(pbase * K_NBR, PW * K_NBR)], idx_v)
    pltpu.sync_copy(am_hbm.at[pl.ds(pbase, PW)], a_v)
    sems = [sem0, sem1]

    def fire(c):
        pltpu.async_copy(
            am_hbm.at[idx_v.at[pl.ds(c * G_IDX, G_IDX)]],
            rows_v.at[c % 2],
            sems[c % 2],
        )

    fire(0)
    for c in range(N_CHUNK):
        if c + 1 < N_CHUNK:
            fire(c + 1)
        pltpu.make_async_copy(
            am_hbm.at[idx_v.at[pl.ds(c * G_IDX, G_IDX)]],
            rows_v.at[c % 2],
            sems[c % 2],
        ).wait()
        buf = c % 2

        def particle(p, carry2, c=c, buf=buf):
            pp = c * P_CHUNK + p
            accs = [jnp.zeros((SC_LANES,), jnp.float32)
                    for _ in range(D_OUT // SC_LANES)]
            for j in range(K_NBR):
                r = p * K_NBR + j
                for q in range(D_OUT // SC_LANES):
                    aq = a_v[pp, pl.ds(q * SC_LANES, SC_LANES)]
                    accs[q] = accs[q] + jnp.maximum(
                        aq + rows_v[buf, r, pl.ds(D_OUT + q * SC_LANES, SC_LANES)],
                        0.0,
                    )
            for q in range(D_OUT // SC_LANES):
                o_v[pp, pl.ds(q * SC_LANES, SC_LANES)] = accs[q] * (1.0 / K_NBR)
            return carry2

        lax.fori_loop(0, P_CHUNK, particle, 0)
    pltpu.sync_copy(o_v, out_hbm.at[pl.ds(pbase, PW)])


def _sc_call_kwargs():
    return dict(
        out_type=jax.ShapeDtypeStruct((N_PTS, D_OUT), jnp.float32),
        mesh=plsc.VectorSubcoreMesh(
            core_axis_name="c", subcore_axis_name="s",
            num_cores=SC_NC, num_subcores=SC_NS,
        ),
        scratch_types=[
            pltpu.VMEM((PW * K_NBR,), jnp.int32),
            pltpu.VMEM((2, G_IDX, 2 * D_OUT), jnp.float32),
            pltpu.VMEM((PW, 2 * D_OUT), jnp.float32),
            pltpu.VMEM((PW, D_OUT), jnp.float32),
            pltpu.SemaphoreType.DMA,
            pltpu.SemaphoreType.DMA,
        ],
    )


def kernel(events, W, b):
    W1 = W[:, :D_FEAT]
    W2 = W[:, D_FEAT:]
    wt = jnp.concatenate([(W1 - W2).T, W2.T], axis=1)
    bf = jnp.concatenate([b, jnp.zeros_like(b)]).reshape(1, 2 * D_OUT)
    coords_t = jnp.transpose(events[:, :, 0:2], (0, 2, 1))

    tc = pl.pallas_call(_tc_body, **_tc_call_kwargs())
    sc = functools.partial(pl.kernel, **_sc_call_kwargs())(_sc_body)

    tc_outs = [tc(events[e], coords_t[e], wt, bf) for e in range(E_EVENTS)]
    outs = [sc(idx.reshape(N_PTS * K_NBR), am) for am, idx in tc_outs]
    return jnp.stack(outs)
